# Initial kernel scaffold; baseline (speedup 1.0000x reference)
import jax
import jax.numpy as jnp
from jax.experimental import pallas as pl


def kernel(tokens, targets):
    raise NotImplementedError("write your pallas kernel here")


# speedup vs baseline: 15.6429x; 15.6429x over previous
import functools

import numpy as np
import jax
import jax.numpy as jnp
from jax import lax
from jax.experimental import pallas as pl
from jax.experimental.pallas import tpu as pltpu
from jax.experimental.pallas import tpu_sc as plsc

_NUM_CLASSES = 1000
_B, _L, _D = 64, 2048, 768

_LAM = float(jax.random.beta(jax.random.split(jax.random.key(42), 3)[1], 1.0, 1.0))
_NUM_MIX = int((1.0 - _LAM) * _L)
_KP, _KB, _KM = jax.random.split(jax.random.key(42), 3)
_PERM = np.asarray(jax.random.permutation(_KP, _B))
_MIX = np.asarray(jax.random.permutation(_KM, _L)[:_NUM_MIX])
_TRUE_LAM = 1.0 - _NUM_MIX / _L

_MASKL = np.zeros(_L, np.bool_)
_MASKL[_MIX] = True
_SRC_B = np.where(_MASKL[None, :], _PERM[:, None], np.arange(_B)[:, None])
_SRC_ROWS = (_SRC_B * _L + np.arange(_L)[None, :]).astype(np.int32).reshape(-1)

_NC, _NS = 2, 16
_NW = _NC * _NS
_ROWS = _B * _L
_RPW = _ROWS // _NW
_CHUNK = 64
_NCHUNK = _RPW // _CHUNK

_sc_mesh = plsc.VectorSubcoreMesh(
    core_axis_name="c", subcore_axis_name="s", num_cores=_NC, num_subcores=_NS
)


@functools.partial(
    pl.kernel,
    out_type=jax.ShapeDtypeStruct((_ROWS, _D), jnp.float32),
    mesh=_sc_mesh,
    scratch_types=[
        pltpu.VMEM((_RPW,), jnp.int32),
        pltpu.VMEM((_CHUNK, _D), jnp.float32),
        pltpu.VMEM((_CHUNK, _D), jnp.float32),
        pltpu.SemaphoreType.DMA,
        pltpu.SemaphoreType.DMA,
    ],
)
def _mix_tokens_sc(tok_hbm, src_hbm, out_hbm, idx_v, buf0, buf1, gsem0, gsem1):
    wid = lax.axis_index("s") * _NC + lax.axis_index("c")
    base = wid * _RPW
    pltpu.sync_copy(src_hbm.at[pl.ds(base, _RPW)], idx_v)

    def start_gather(g, buf, sem):
        pltpu.async_copy(tok_hbm.at[idx_v.at[pl.ds(g * _CHUNK, _CHUNK)]], buf, sem)

    def wait_gather(buf, sem):
        pltpu.make_async_copy(tok_hbm.at[pl.ds(0, _CHUNK)], buf, sem).wait()

    def write(g, buf):
        pltpu.sync_copy(buf, out_hbm.at[pl.ds(base + g * _CHUNK, _CHUNK)])

    def pair(i, carry, last):
        g0 = 2 * i
        start_gather(g0 + 1, buf1, gsem1)
        wait_gather(buf0, gsem0)
        write(g0, buf0)
        if not last:
            start_gather(g0 + 2, buf0, gsem0)
        wait_gather(buf1, gsem1)
        write(g0 + 1, buf1)
        return carry

    start_gather(0, buf0, gsem0)
    lax.fori_loop(0, _NCHUNK // 2 - 1, lambda i, c: pair(i, c, False), 0)
    pair(_NCHUNK // 2 - 1, 0, True)


_CPAD = 1024


def _targets_body(ta_ref, tb_ref, out_ref):
    cls = lax.broadcasted_iota(jnp.int32, (_B, _CPAD), 1)
    oa = (cls == ta_ref[...]).astype(jnp.float32)
    ob = (cls == tb_ref[...]).astype(jnp.float32)
    out_ref[...] = _TRUE_LAM * oa + (1.0 - _TRUE_LAM) * ob


_targets_call = pl.pallas_call(
    _targets_body,
    out_shape=jax.ShapeDtypeStruct((_B, _CPAD), jnp.float32),
)


def kernel(tokens, targets):
    tok2d = tokens.reshape(_ROWS, _D)
    src = jnp.asarray(_SRC_ROWS)
    if _NUM_MIX == 0:
        tokens_mixed = tokens
    else:
        tokens_mixed = _mix_tokens_sc(tok2d, src).reshape(_B, _L, _D)
    t32 = targets.astype(jnp.int32)
    ta = t32.reshape(_B, 1)
    tb = t32[jnp.asarray(_PERM)].reshape(_B, 1)
    mixed_targets = _targets_call(ta, tb)[:, :_NUM_CLASSES]
    return tokens_mixed, mixed_targets
